# baseline (speedup 1.0000x reference)
import math

import jax
import jax.numpy as jnp
from jax import lax
from jax.experimental import pallas as pl
from jax.experimental.pallas import tpu as pltpu
from jax.experimental.pallas import tpu_sc as plsc

VOCAB = 1000000
EMBED = 64
WINDOW = 200
BATCH = 4096

NC, NS, LANES = 2, 16, 16
NW = NC * NS
SEQ_PER_W = BATCH // NW
ROWS = WINDOW
VECS_PER_ROW = EMBED // LANES
SCALE = math.sqrt(EMBED)


def _body(x_hbm, table_hbm, pe_hbm, out_hbm, idx_v, rows_v, pe_v, sem):
    wid = lax.axis_index("s") * NC + lax.axis_index("c")

    pltpu.sync_copy(pe_hbm, pe_v)

    def seq_step(i, _):
        base = (wid * SEQ_PER_W + i) * WINDOW
        pltpu.sync_copy(x_hbm.at[pl.ds(base, ROWS)], idx_v)
        pltpu.async_copy(table_hbm.at[idx_v], rows_v, sem).wait()

        def row_step(r, _):
            for k in range(VECS_PER_ROW):
                sl = pl.ds(k * LANES, LANES)
                rows_v[r, sl] = rows_v[r, sl] * SCALE + pe_v[r, sl]
            return ()

        lax.fori_loop(0, ROWS, row_step, (), unroll=False)
        pltpu.sync_copy(rows_v, out_hbm.at[pl.ds(base, ROWS)])
        return ()

    lax.fori_loop(0, SEQ_PER_W, seq_step, (), unroll=False)


def kernel(x, table, pos_encoding):
    xf = x.reshape(BATCH * WINDOW)
    pe = pos_encoding[:WINDOW, :]

    mesh = plsc.VectorSubcoreMesh(
        core_axis_name="c", subcore_axis_name="s",
        num_cores=NC, num_subcores=NS)

    out = pl.kernel(
        _body,
        out_type=jax.ShapeDtypeStruct((BATCH * WINDOW, EMBED), jnp.float32),
        mesh=mesh,
        scratch_types=[
            pltpu.VMEM((ROWS,), jnp.int32),
            pltpu.VMEM((ROWS, EMBED), jnp.float32),
            pltpu.VMEM((WINDOW, EMBED), jnp.float32),
            pltpu.SemaphoreType.DMA,
        ],
        compiler_params=pltpu.CompilerParams(use_tc_tiling_on_sc=False),
    )(xf, table, pe)
    return out.reshape(BATCH, WINDOW, EMBED)
